# baseline (speedup 1.0000x reference)
import functools

import jax
import jax.numpy as jnp
from jax import lax
from jax.experimental import pallas as pl
from jax.experimental.pallas import tpu as pltpu
from jax.experimental.pallas import tpu_sc as plsc

BLK = 64


def _permute_rows(table, idx):
    T, H = table.shape
    info = plsc.get_sparse_core_info()
    nw = info.num_cores * info.num_subcores
    b_per_w = T // nw
    mesh = plsc.VectorSubcoreMesh(core_axis_name="c", subcore_axis_name="s")

    @functools.partial(
        pl.kernel, mesh=mesh,
        out_type=jax.ShapeDtypeStruct((T, H), jnp.float32),
        scratch_types=[
            pltpu.VMEM((b_per_w,), jnp.int32),
            pltpu.VMEM((b_per_w, H), jnp.float32),
            pltpu.SemaphoreType.DMA,
        ],
    )
    def gather_k(table_hbm, idx_hbm, out_hbm, idx_v, rows_v, sem):
        wid = lax.axis_index("s") * info.num_cores + lax.axis_index("c")
        base = wid * b_per_w
        pltpu.sync_copy(idx_hbm.at[pl.ds(base, b_per_w)], idx_v)
        pltpu.async_copy(table_hbm.at[idx_v], rows_v, sem).wait()
        pltpu.sync_copy(rows_v, out_hbm.at[pl.ds(base, b_per_w)])

    return gather_k(table, idx)


def _body(s_ref, nb_ref, off_ref, end_ref, x_ref, wa_ref, wb_ref, wc_ref,
          o_ref):
    e = pl.program_id(0)
    T = o_ref.shape[0]
    HO = wa_ref.shape[1]

    @pl.when(e == 0)
    def _init():
        o_ref[...] = jnp.zeros_like(o_ref)

    off = off_ref[e]
    end = end_ref[e]
    s0 = s_ref[e]

    def loop(i, carry):
        lo = s0 + i * BLK
        s = pl.multiple_of(jnp.minimum(lo, T - BLK), 8)
        rows = x_ref[pl.ds(s, BLK), :]
        pos = s + jax.lax.broadcasted_iota(jnp.int32, (BLK, 1), 0)
        mask = ((pos >= jnp.maximum(off, lo)) &
                (pos < jnp.minimum(end, lo + BLK)))
        xm = jnp.where(mask, rows, 0.0)
        for j, w_ref in enumerate((wa_ref, wb_ref, wc_ref)):
            c = jax.lax.dot_general(
                xm, w_ref[0], (((1,), (1,)), ((), ())),
                preferred_element_type=jnp.float32)
            o_ref[pl.ds(s, BLK), j * HO:(j + 1) * HO] += c
        return carry

    jax.lax.fori_loop(0, nb_ref[e], loop, 0)


def kernel(x, indices, control_vectors):
    T, H = x.shape
    E = control_vectors.shape[0]

    iota = jnp.arange(T, dtype=jnp.int32)
    se, sort_idx = jax.lax.sort((indices, iota), num_keys=1)
    inv = jnp.zeros((T,), jnp.int32).at[sort_idx].set(iota)
    cnt = jnp.zeros((E,), jnp.int32).at[indices].add(1)
    end = jnp.cumsum(cnt)
    off = end - cnt
    s0 = (off // 8) * 8
    nblk = jnp.where(cnt > 0, (end - s0 + BLK - 1) // BLK, 0).astype(jnp.int32)
    off = off.astype(jnp.int32)
    end = end.astype(jnp.int32)
    s0 = s0.astype(jnp.int32)

    x_sorted = _permute_rows(x, sort_idx)

    grid_spec = pltpu.PrefetchScalarGridSpec(
        num_scalar_prefetch=4,
        grid=(E,),
        in_specs=[
            pl.BlockSpec((T, H), lambda e, *_: (0, 0)),
            pl.BlockSpec((1, H // 3, H), lambda e, *_: (e, 0, 0)),
            pl.BlockSpec((1, H // 3, H), lambda e, *_: (e, 1, 0)),
            pl.BlockSpec((1, H // 3, H), lambda e, *_: (e, 2, 0)),
        ],
        out_specs=pl.BlockSpec((T, H), lambda e, *_: (0, 0)),
    )
    out_sorted = pl.pallas_call(
        _body,
        grid_spec=grid_spec,
        out_shape=jax.ShapeDtypeStruct((T, H), jnp.float32),
        compiler_params=pltpu.CompilerParams(
            dimension_semantics=("arbitrary",)),
    )(s0, nblk, off, end, x_sorted, control_vectors, control_vectors,
      control_vectors)

    return _permute_rows(out_sorted, inv)


# speedup vs baseline: 1.0066x; 1.0066x over previous
import functools

import jax
import jax.numpy as jnp
from jax import lax
from jax.experimental import pallas as pl
from jax.experimental.pallas import tpu as pltpu
from jax.experimental.pallas import tpu_sc as plsc

BLK = 64


def _permute_rows(table, idx):
    T, H = table.shape
    info = plsc.get_sparse_core_info()
    nw = info.num_cores * info.num_subcores
    b_per_w = T // nw
    mesh = plsc.VectorSubcoreMesh(core_axis_name="c", subcore_axis_name="s")

    @functools.partial(
        pl.kernel, mesh=mesh,
        out_type=jax.ShapeDtypeStruct((T, H), jnp.float32),
        scratch_types=[
            pltpu.VMEM((b_per_w,), jnp.int32),
            pltpu.VMEM((b_per_w, H), jnp.float32),
            pltpu.SemaphoreType.DMA,
        ],
    )
    def gather_k(table_hbm, idx_hbm, out_hbm, idx_v, rows_v, sem):
        wid = lax.axis_index("s") * info.num_cores + lax.axis_index("c")
        base = wid * b_per_w
        pltpu.sync_copy(idx_hbm.at[pl.ds(base, b_per_w)], idx_v)
        pltpu.async_copy(table_hbm.at[idx_v], rows_v, sem).wait()
        pltpu.sync_copy(rows_v, out_hbm.at[pl.ds(base, b_per_w)])

    return gather_k(table, idx)


def _body(s_ref, nb_ref, off_ref, end_ref, x_ref, wa_ref, wb_ref, o_ref):
    e = pl.program_id(0)
    T = o_ref.shape[0]
    HO = wa_ref.shape[1]

    @pl.when(e == 0)
    def _init():
        o_ref[...] = jnp.zeros_like(o_ref)

    off = off_ref[e]
    end = end_ref[e]
    s0 = s_ref[e]

    def loop(i, carry):
        lo = s0 + i * BLK
        s = pl.multiple_of(jnp.minimum(lo, T - BLK), 8)
        rows = x_ref[pl.ds(s, BLK), :]
        pos = s + jax.lax.broadcasted_iota(jnp.int32, (BLK, 1), 0)
        mask = ((pos >= jnp.maximum(off, lo)) &
                (pos < jnp.minimum(end, lo + BLK)))
        xm = jnp.where(mask, rows, 0.0)
        for j, w_ref in enumerate((wa_ref, wb_ref)):
            c = jax.lax.dot_general(
                xm, w_ref[0], (((1,), (1,)), ((), ())),
                preferred_element_type=jnp.float32)
            o_ref[pl.ds(s, BLK), j * HO:(j + 1) * HO] += c
        return carry

    jax.lax.fori_loop(0, nb_ref[e], loop, 0)


def kernel(x, indices, control_vectors):
    T, H = x.shape
    E = control_vectors.shape[0]

    iota = jnp.arange(T, dtype=jnp.int32)
    se, sort_idx = jax.lax.sort((indices, iota), num_keys=1)
    inv = jnp.zeros((T,), jnp.int32).at[sort_idx].set(iota)
    cnt = jnp.zeros((E,), jnp.int32).at[indices].add(1)
    end = jnp.cumsum(cnt)
    off = end - cnt
    s0 = (off // 8) * 8
    nblk = jnp.where(cnt > 0, (end - s0 + BLK - 1) // BLK, 0).astype(jnp.int32)
    off = off.astype(jnp.int32)
    end = end.astype(jnp.int32)
    s0 = s0.astype(jnp.int32)

    x_sorted = _permute_rows(x, sort_idx)

    grid_spec = pltpu.PrefetchScalarGridSpec(
        num_scalar_prefetch=4,
        grid=(E,),
        in_specs=[
            pl.BlockSpec((T, H), lambda e, *_: (0, 0)),
            pl.BlockSpec((1, H // 2, H), lambda e, *_: (e, 0, 0)),
            pl.BlockSpec((1, H // 2, H), lambda e, *_: (e, 1, 0)),
        ],
        out_specs=pl.BlockSpec((T, H), lambda e, *_: (0, 0)),
    )
    out_sorted = pl.pallas_call(
        _body,
        grid_spec=grid_spec,
        out_shape=jax.ShapeDtypeStruct((T, H), jnp.float32),
        compiler_params=pltpu.CompilerParams(
            dimension_semantics=("arbitrary",)),
    )(s0, nblk, off, end, x_sorted, control_vectors, control_vectors)

    return _permute_rows(out_sorted, inv)
